# baseline (speedup 1.0000x reference)
import functools

import jax
import jax.numpy as jnp
from jax import lax
from jax.experimental import pallas as pl
from jax.experimental.pallas import tpu as pltpu
from jax.experimental.pallas import tpu_sc as plsc

_B = 16384
_E = 64
_NU = 1000000
_NM = 100000
_BU = _NU // 128
_BM = _NM // 128
_UTAIL = _NU - _BU * 128
_MTAIL = _NM - _BM * 128
_CHW = 1024
_HCAP = 768
_NHV = _HCAP // 16
_TCAP = 64
_MCAP = 256
_CHSH = 10
_NCNT = 80
_GCAP = 1728
_NSTG = 8
_DUMMY = _B
_OUTR = _B + 16
_MLP_BLK = 2048
_NPIECE = 8
_PIECE = _B // _NPIECE


def _iota16():
    return lax.iota(jnp.int32, 16)


def _full16(x):
    return jnp.full((16,), x, jnp.int32)


def _sc_body(nc, users_h, movies_h, ut_h, mt_h, utail_h, mtail_h,
             uout, mout,
             ibuf, locl, posl, tlocl, tposl, mrel, mpos, scnt, sloc, spos,
             buf, tailu, tailm, stg, posr, sem, semo):
    wid = lax.axis_index("s") * nc + lax.axis_index("c")

    def drain_one():
        pltpu.make_async_copy(
            uout.at[pl.ds(0, 16)], stg.at[pl.ds(0, 16)], semo).wait()

    def prepass(idx_h, col_start, ncols, dst_loc, dst_pos):
        def piece(p, nhit):
            pltpu.sync_copy(idx_h.at[pl.ds(p * _PIECE, _PIECE)], ibuf)

            def vbody(v, nh):
                vec = ibuf[pl.ds(v * 16, 16)]
                loc = vec - _full16(col_start)
                zero = _full16(0)
                m = (loc >= zero) & (loc < _full16(ncols))
                slots = _full16(nh - 1) + plsc.cumsum(
                    jnp.where(m, _full16(1), zero))
                plsc.store_scatter(dst_loc, [slots], loc, mask=m)
                pos = _full16(p * _PIECE + v * 16) + _iota16()
                plsc.store_scatter(dst_pos, [slots], pos, mask=m)
                return slots[15] + 1

            return lax.fori_loop(0, _PIECE // 16, vbody, nhit)

        nhit = 0
        for p in range(_NPIECE):
            nhit = piece(p, nhit)
        return nhit

    def flush_group(src_buf, dst_out, rel, pos, valid, nf):
        @pl.when(nf >= _NSTG)
        def _():
            drain_one()
        slot = lax.rem(nf, _NSTG)
        srow = pl.multiple_of(slot * 16, 16)
        mi = jnp.where(valid, _full16(1), _full16(0))
        for j in range(16):
            @pl.when(mi[j] == 1)
            def _(j=j):
                col = _full16(rel[j])
                for f in range(4):
                    vals = plsc.load_gather(
                        src_buf, [_iota16() + _full16(f * 16), col])
                    stg[srow + j, pl.ds(f * 16, 16)] = vals
        possel = jnp.where(valid, pos, _full16(_DUMMY))
        posr[slot, pl.ds(0, 16)] = possel
        pltpu.async_copy(
            stg.at[pl.ds(srow, 16)], dst_out.at[posr.at[slot]], semo)
        return nf + 1

    def bucketize(loc_ref, pos_ref, nhit):
        nhv_d = (nhit + 15) // 16
        ones = _full16(1)
        zero = _full16(0)
        for k in range(_NCNT // 16):
            scnt[pl.ds(k * 16, 16)] = zero
        for k in range(_GCAP // 16):
            sloc[pl.ds(k * 16, 16)] = _full16(-1)

        def cnt(h, _):
            lm = loc_ref[pl.ds(h * 16, 16)]
            vmask = (_full16(h * 16) + _iota16()) < _full16(nhit)
            cid = jnp.where(vmask, lax.shift_right_logical(lm, _CHSH), zero)
            plsc.addupdate_scatter(scnt, [cid], ones, mask=vmask)
            return 0

        lax.fori_loop(0, nhv_d, cnt, 0)
        carry = 0
        for k in range(_NCNT // 16):
            c = scnt[pl.ds(k * 16, 16)]
            cpad = jnp.bitwise_and(c + _full16(15), _full16(-16))
            inc = plsc.cumsum(cpad) + _full16(carry)
            scnt[pl.ds(k * 16, 16)] = inc - cpad
            carry = inc[15]

        def place(h, _):
            lm = loc_ref[pl.ds(h * 16, 16)]
            pm = pos_ref[pl.ds(h * 16, 16)]
            vmask = (_full16(h * 16) + _iota16()) < _full16(nhit)
            cid = jnp.where(vmask, lax.shift_right_logical(lm, _CHSH), zero)
            base = plsc.load_gather(scnt, [cid])
            occ, _ = plsc.scan_count(cid, vmask)
            slot = base + occ
            plsc.store_scatter(sloc, [slot], lm, mask=vmask)
            plsc.store_scatter(spos, [slot], pm, mask=vmask)
            plsc.addupdate_scatter(scnt, [cid], ones, mask=vmask)
            return 0

        lax.fori_loop(0, nhv_d, place, 0)
        return carry // 16

    def extract_grouped(t_h, dst_out, col_start, ncols, ngrp, nflush):
        col_end = col_start + ncols

        def grp(g, carry):
            nf, cur = carry
            loc16 = sloc[pl.ds(g * 16, 16)]
            pos16 = spos[pl.ds(g * 16, 16)]
            valid = loc16 >= _full16(0)
            cidv = jnp.where(valid, lax.shift_right_logical(loc16, _CHSH),
                             _full16(0))
            cidg = jnp.max(cidv)

            @pl.when(cidg != cur)
            def _():
                cs = jnp.minimum(col_start + cidg * _CHW, col_end - _CHW)
                cs = pl.multiple_of(cs, 128)
                copies = [
                    pltpu.async_copy(
                        t_h.at[pl.ds(gg * 8, 8), pl.ds(cs, _CHW)],
                        buf.at[pl.ds(gg * 8, 8)], sem)
                    for gg in range(8)
                ]
                for cp in copies:
                    cp.wait()

            csrel = jnp.minimum(cidg * _CHW, ncols - _CHW)
            rel = loc16 - _full16(csrel)
            nf = flush_group(buf, dst_out, rel, pos16, valid, nf)
            return (nf, cidg)

        nf, _ = lax.fori_loop(0, ngrp, grp, (nflush, -1))
        return nf

    def extract(src_buf, dst_out, loc_ref, pos_ref, nhit, rel0, width,
                nhv, nflush):
        nhv_d = jnp.minimum((nhit + 15) // 16, nhv)

        def scan(h, nm):
            lm = loc_ref[pl.ds(h * 16, 16)]
            pm = pos_ref[pl.ds(h * 16, 16)]
            vmask = (_full16(h * 16) + _iota16()) < _full16(nhit)
            rel = lm - _full16(rel0)
            zero = _full16(0)
            m = vmask & (rel >= zero) & (rel < _full16(width))
            slots = _full16(nm - 1) + plsc.cumsum(
                jnp.where(m, _full16(1), zero))
            plsc.store_scatter(mrel, [slots], rel, mask=m)
            plsc.store_scatter(mpos, [slots], pm, mask=m)
            return nm + plsc.all_reduce_population_count(m)[0]

        nm = lax.fori_loop(0, nhv_d, scan, 0)
        ngrp = (nm + 15) // 16

        def grp(g, nf):
            rel = mrel[pl.ds(g * 16, 16)]
            pos = mpos[pl.ds(g * 16, 16)]
            valid = (_full16(g * 16) + _iota16()) < _full16(nm)
            return flush_group(src_buf, dst_out, rel, pos, valid, nf)

        return lax.fori_loop(0, ngrp, grp, nflush)

    def run_table(idx_h, t_h, out_h, nblocks, rem, nflush):
        nb = jnp.int32(nblocks) + (wid < rem).astype(jnp.int32)
        bstart = wid * nblocks + jnp.minimum(wid, rem)
        col_start = bstart * 128
        ncols = nb * 128
        col_end = col_start + ncols
        nhit = prepass(idx_h, col_start, ncols, locl, posl)
        ngrp = bucketize(locl, posl, nhit)
        return extract_grouped(t_h, out_h, col_start, ncols, ngrp, nflush)

    def run_tail(idx_h, tail_h, tail_buf, out_h, tail_base, ntail, nflush):
        pltpu.sync_copy(tail_h, tail_buf)
        nhit = prepass(idx_h, tail_base, ntail, tlocl, tposl)
        return extract(tail_buf, out_h, tlocl, tposl, nhit, 0, ntail,
                       _TCAP // 16, nflush)

    nflush = 0
    nflush = run_table(users_h, ut_h, uout, _BU // 32, _BU % 32, nflush)
    nflush = run_table(movies_h, mt_h, mout, _BM // 32, _BM % 32, nflush)

    def tails(nf):
        nf = run_tail(users_h, utail_h, tailu, uout, _BU * 128, _UTAIL, nf)
        nf = run_tail(movies_h, mtail_h, tailm, mout, _BM * 128, _MTAIL, nf)
        return nf

    nflush = lax.cond(wid == 31, tails, lambda nf: nf, nflush)

    def fin(i, _):
        drain_one()
        return 0

    lax.fori_loop(0, jnp.minimum(nflush, _NSTG), fin, 0)


def _sc_gather(users, movies, ut, mt, utail, mtail):
    info = plsc.get_sparse_core_info()
    nc = info.num_cores
    mesh = plsc.VectorSubcoreMesh(core_axis_name="c", subcore_axis_name="s")
    k = pl.kernel(
        functools.partial(_sc_body, nc),
        out_type=(jax.ShapeDtypeStruct((_OUTR, 128), jnp.float32),
                  jax.ShapeDtypeStruct((_OUTR, 128), jnp.float32)),
        mesh=mesh,
        scratch_types=[
            pltpu.VMEM((_PIECE,), jnp.int32),
            pltpu.VMEM((_HCAP + 16,), jnp.int32),
            pltpu.VMEM((_HCAP + 16,), jnp.int32),
            pltpu.VMEM((_TCAP + 16,), jnp.int32),
            pltpu.VMEM((_TCAP + 16,), jnp.int32),
            pltpu.VMEM((_MCAP + 16,), jnp.int32),
            pltpu.VMEM((_MCAP + 16,), jnp.int32),
            pltpu.VMEM((_NCNT,), jnp.int32),
            pltpu.VMEM((_GCAP,), jnp.int32),
            pltpu.VMEM((_GCAP,), jnp.int32),
            pltpu.VMEM((_E, _CHW), jnp.float32),
            pltpu.VMEM((_E, _UTAIL), jnp.float32),
            pltpu.VMEM((_E, _MTAIL), jnp.float32),
            pltpu.VMEM((_NSTG * 16, 128), jnp.float32),
            pltpu.VMEM((_NSTG, 16), jnp.int32),
            pltpu.SemaphoreType.DMA,
            pltpu.SemaphoreType.DMA,
        ],
        compiler_params=pltpu.CompilerParams(needs_layout_passes=False),
    )
    return k(users, movies, ut, mt, utail, mtail)


def _mlp_body(u_ref, m_ref, w1u_ref, w1m_ref, b1_ref, w2_ref, b2_ref,
              w3_ref, b3_ref, out_ref):
    ue = u_ref[...][:, :_E]
    me = m_ref[...][:, :_E]
    h = jnp.dot(ue, w1u_ref[...], preferred_element_type=jnp.float32)
    h = h + jnp.dot(me, w1m_ref[...], preferred_element_type=jnp.float32)
    h = jnp.maximum(h + b1_ref[...], 0.0)
    h = jnp.maximum(
        jnp.dot(h, w2_ref[...], preferred_element_type=jnp.float32)
        + b2_ref[...], 0.0)
    o = jnp.dot(h, w3_ref[...], preferred_element_type=jnp.float32) + b3_ref[...]
    out_ref[...] = jnp.maximum(o[:, 0], 0.0)


def _mlp(u_rows, m_rows, W1, b1, W2, b2, W3, b3):
    w1u, w1m = W1[:_E], W1[_E:]
    grid = _B // _MLP_BLK
    row_spec = pl.BlockSpec((_MLP_BLK, 128), lambda i: (i, 0))

    def full(shape):
        return pl.BlockSpec(shape, lambda i: (0, 0))

    return pl.pallas_call(
        _mlp_body,
        grid=(grid,),
        in_specs=[row_spec, row_spec, full((_E, 64)), full((_E, 64)),
                  full((1, 64)), full((64, 16)), full((1, 16)),
                  full((16, 1)), full((1, 1))],
        out_specs=pl.BlockSpec((_MLP_BLK,), lambda i: (i,)),
        out_shape=jax.ShapeDtypeStruct((_B,), jnp.float32),
    )(u_rows, m_rows, w1u, w1m, b1.reshape(1, -1), W2, b2.reshape(1, -1),
      W3, b3.reshape(1, -1))


def kernel(users, movies, emb_users, emb_movies, W1, b1, W2, b2, W3, b3):
    users = users.astype(jnp.int32)
    movies = movies.astype(jnp.int32)
    ut = emb_users.T
    mt = emb_movies.T
    utail = ut[:, _BU * 128:]
    mtail = mt[:, _BM * 128:]
    u_rows, m_rows = _sc_gather(users, movies, ut, mt, utail, mtail)
    return _mlp(u_rows, m_rows, W1, b1, W2, b2, W3, b3)


# speedup vs baseline: 1.0548x; 1.0548x over previous
import functools

import jax
import jax.numpy as jnp
from jax import lax
from jax.experimental import pallas as pl
from jax.experimental.pallas import tpu as pltpu
from jax.experimental.pallas import tpu_sc as plsc

_B = 16384
_E = 64
_NU = 1000000
_NM = 100000
_BU = _NU // 128
_BM = _NM // 128
_UTAIL = _NU - _BU * 128
_MTAIL = _NM - _BM * 128
_CHW = 1024
_HCAP = 768
_NHV = _HCAP // 16
_TCAP = 64
_MCAP = 256
_CHSH = 10
_NCNT = 80
_GCAP = 1728
_NSTG = 16
_DUMMY = _B
_OUTR = _B + 16
_MLP_BLK = 2048
_NPIECE = 8
_PIECE = _B // _NPIECE


def _iota16():
    return lax.iota(jnp.int32, 16)


def _full16(x):
    return jnp.full((16,), x, jnp.int32)


def _sc_body(nc, users_h, movies_h, ut_h, mt_h, utail_h, mtail_h,
             uout, mout,
             ibuf, locl, posl, tlocl, tposl, mrel, mpos, scnt, sloc, spos,
             buf, tailu, tailm, stg, posr, sem, semo):
    wid = lax.axis_index("s") * nc + lax.axis_index("c")

    def drain_one():
        pltpu.make_async_copy(
            uout.at[pl.ds(0, 16)], stg.at[pl.ds(0, 16)], semo).wait()

    def prepass(idx_h, col_start, ncols, dst_loc, dst_pos):
        def piece(p, nhit):
            pltpu.sync_copy(idx_h.at[pl.ds(p * _PIECE, _PIECE)], ibuf)

            def vbody(v, nh):
                vec = ibuf[pl.ds(v * 16, 16)]
                loc = vec - _full16(col_start)
                zero = _full16(0)
                m = (loc >= zero) & (loc < _full16(ncols))
                slots = _full16(nh - 1) + plsc.cumsum(
                    jnp.where(m, _full16(1), zero))
                plsc.store_scatter(dst_loc, [slots], loc, mask=m)
                pos = _full16(p * _PIECE + v * 16) + _iota16()
                plsc.store_scatter(dst_pos, [slots], pos, mask=m)
                return slots[15] + 1

            return lax.fori_loop(0, _PIECE // 16, vbody, nhit)

        nhit = 0
        for p in range(_NPIECE):
            nhit = piece(p, nhit)
        return nhit

    def flush_group(src_buf, dst_out, rel, pos, valid, nf):
        @pl.when(nf >= _NSTG)
        def _():
            drain_one()
        slot = lax.rem(nf, _NSTG)
        srow = pl.multiple_of(slot * 16, 16)
        mi = jnp.where(valid, _full16(1), _full16(0))
        for j in range(16):
            @pl.when(mi[j] == 1)
            def _(j=j):
                col = _full16(rel[j])
                for f in range(4):
                    vals = plsc.load_gather(
                        src_buf, [_iota16() + _full16(f * 16), col])
                    stg[srow + j, pl.ds(f * 16, 16)] = vals
        possel = jnp.where(valid, pos, _full16(_DUMMY))
        posr[slot, pl.ds(0, 16)] = possel
        pltpu.async_copy(
            stg.at[pl.ds(srow, 16)], dst_out.at[posr.at[slot]], semo)
        return nf + 1

    def bucketize(loc_ref, pos_ref, nhit):
        nhv_d = (nhit + 15) // 16
        ones = _full16(1)
        zero = _full16(0)
        for k in range(_NCNT // 16):
            scnt[pl.ds(k * 16, 16)] = zero
        for k in range(_GCAP // 16):
            sloc[pl.ds(k * 16, 16)] = _full16(-1)

        def cnt(h, _):
            lm = loc_ref[pl.ds(h * 16, 16)]
            vmask = (_full16(h * 16) + _iota16()) < _full16(nhit)
            cid = jnp.where(vmask, lax.shift_right_logical(lm, _CHSH), zero)
            plsc.addupdate_scatter(scnt, [cid], ones, mask=vmask)
            return 0

        lax.fori_loop(0, nhv_d, cnt, 0)
        carry = 0
        for k in range(_NCNT // 16):
            c = scnt[pl.ds(k * 16, 16)]
            cpad = jnp.bitwise_and(c + _full16(15), _full16(-16))
            inc = plsc.cumsum(cpad) + _full16(carry)
            scnt[pl.ds(k * 16, 16)] = inc - cpad
            carry = inc[15]

        def place(h, _):
            lm = loc_ref[pl.ds(h * 16, 16)]
            pm = pos_ref[pl.ds(h * 16, 16)]
            vmask = (_full16(h * 16) + _iota16()) < _full16(nhit)
            cid = jnp.where(vmask, lax.shift_right_logical(lm, _CHSH), zero)
            base = plsc.load_gather(scnt, [cid])
            occ, _ = plsc.scan_count(cid, vmask)
            slot = base + occ
            plsc.store_scatter(sloc, [slot], lm, mask=vmask)
            plsc.store_scatter(spos, [slot], pm, mask=vmask)
            plsc.addupdate_scatter(scnt, [cid], ones, mask=vmask)
            return 0

        lax.fori_loop(0, nhv_d, place, 0)
        return carry // 16

    def extract_grouped(t_h, dst_out, col_start, ncols, ngrp, nflush):
        col_end = col_start + ncols

        def grp(g, carry):
            nf, cur = carry
            loc16 = sloc[pl.ds(g * 16, 16)]
            pos16 = spos[pl.ds(g * 16, 16)]
            valid = loc16 >= _full16(0)
            cidv = jnp.where(valid, lax.shift_right_logical(loc16, _CHSH),
                             _full16(0))
            cidg = jnp.max(cidv)

            @pl.when(cidg != cur)
            def _():
                cs = jnp.minimum(col_start + cidg * _CHW, col_end - _CHW)
                cs = pl.multiple_of(cs, 128)
                copies = [
                    pltpu.async_copy(
                        t_h.at[pl.ds(gg * 8, 8), pl.ds(cs, _CHW)],
                        buf.at[pl.ds(gg * 8, 8)], sem)
                    for gg in range(8)
                ]
                for cp in copies:
                    cp.wait()

            csrel = jnp.minimum(cidg * _CHW, ncols - _CHW)
            rel = loc16 - _full16(csrel)
            nf = flush_group(buf, dst_out, rel, pos16, valid, nf)
            return (nf, cidg)

        nf, _ = lax.fori_loop(0, ngrp, grp, (nflush, -1))
        return nf

    def extract(src_buf, dst_out, loc_ref, pos_ref, nhit, rel0, width,
                nhv, nflush):
        nhv_d = jnp.minimum((nhit + 15) // 16, nhv)

        def scan(h, nm):
            lm = loc_ref[pl.ds(h * 16, 16)]
            pm = pos_ref[pl.ds(h * 16, 16)]
            vmask = (_full16(h * 16) + _iota16()) < _full16(nhit)
            rel = lm - _full16(rel0)
            zero = _full16(0)
            m = vmask & (rel >= zero) & (rel < _full16(width))
            slots = _full16(nm - 1) + plsc.cumsum(
                jnp.where(m, _full16(1), zero))
            plsc.store_scatter(mrel, [slots], rel, mask=m)
            plsc.store_scatter(mpos, [slots], pm, mask=m)
            return nm + plsc.all_reduce_population_count(m)[0]

        nm = lax.fori_loop(0, nhv_d, scan, 0)
        ngrp = (nm + 15) // 16

        def grp(g, nf):
            rel = mrel[pl.ds(g * 16, 16)]
            pos = mpos[pl.ds(g * 16, 16)]
            valid = (_full16(g * 16) + _iota16()) < _full16(nm)
            return flush_group(src_buf, dst_out, rel, pos, valid, nf)

        return lax.fori_loop(0, ngrp, grp, nflush)

    def run_table(idx_h, t_h, out_h, nblocks, rem, nflush):
        nb = jnp.int32(nblocks) + (wid < rem).astype(jnp.int32)
        bstart = wid * nblocks + jnp.minimum(wid, rem)
        col_start = bstart * 128
        ncols = nb * 128
        col_end = col_start + ncols
        nhit = prepass(idx_h, col_start, ncols, locl, posl)
        ngrp = bucketize(locl, posl, nhit)
        return extract_grouped(t_h, out_h, col_start, ncols, ngrp, nflush)

    def run_tail(idx_h, tail_h, tail_buf, out_h, tail_base, ntail, nflush):
        pltpu.sync_copy(tail_h, tail_buf)
        nhit = prepass(idx_h, tail_base, ntail, tlocl, tposl)
        return extract(tail_buf, out_h, tlocl, tposl, nhit, 0, ntail,
                       _TCAP // 16, nflush)

    nflush = 0
    nflush = run_table(users_h, ut_h, uout, _BU // 32, _BU % 32, nflush)
    nflush = run_table(movies_h, mt_h, mout, _BM // 32, _BM % 32, nflush)

    def tails(nf):
        nf = run_tail(users_h, utail_h, tailu, uout, _BU * 128, _UTAIL, nf)
        nf = run_tail(movies_h, mtail_h, tailm, mout, _BM * 128, _MTAIL, nf)
        return nf

    nflush = lax.cond(wid == 31, tails, lambda nf: nf, nflush)

    def fin(i, _):
        drain_one()
        return 0

    lax.fori_loop(0, jnp.minimum(nflush, _NSTG), fin, 0)


def _sc_gather(users, movies, ut, mt, utail, mtail):
    info = plsc.get_sparse_core_info()
    nc = info.num_cores
    mesh = plsc.VectorSubcoreMesh(core_axis_name="c", subcore_axis_name="s")
    k = pl.kernel(
        functools.partial(_sc_body, nc),
        out_type=(jax.ShapeDtypeStruct((_OUTR, 128), jnp.float32),
                  jax.ShapeDtypeStruct((_OUTR, 128), jnp.float32)),
        mesh=mesh,
        scratch_types=[
            pltpu.VMEM((_PIECE,), jnp.int32),
            pltpu.VMEM((_HCAP + 16,), jnp.int32),
            pltpu.VMEM((_HCAP + 16,), jnp.int32),
            pltpu.VMEM((_TCAP + 16,), jnp.int32),
            pltpu.VMEM((_TCAP + 16,), jnp.int32),
            pltpu.VMEM((_MCAP + 16,), jnp.int32),
            pltpu.VMEM((_MCAP + 16,), jnp.int32),
            pltpu.VMEM((_NCNT,), jnp.int32),
            pltpu.VMEM((_GCAP,), jnp.int32),
            pltpu.VMEM((_GCAP,), jnp.int32),
            pltpu.VMEM((_E, _CHW), jnp.float32),
            pltpu.VMEM((_E, _UTAIL), jnp.float32),
            pltpu.VMEM((_E, _MTAIL), jnp.float32),
            pltpu.VMEM((_NSTG * 16, 128), jnp.float32),
            pltpu.VMEM((_NSTG, 16), jnp.int32),
            pltpu.SemaphoreType.DMA,
            pltpu.SemaphoreType.DMA,
        ],
        compiler_params=pltpu.CompilerParams(needs_layout_passes=False),
    )
    return k(users, movies, ut, mt, utail, mtail)


def _mlp_body(u_ref, m_ref, w1u_ref, w1m_ref, b1_ref, w2_ref, b2_ref,
              w3_ref, b3_ref, out_ref):
    ue = u_ref[...][:, :_E]
    me = m_ref[...][:, :_E]
    h = jnp.dot(ue, w1u_ref[...], preferred_element_type=jnp.float32)
    h = h + jnp.dot(me, w1m_ref[...], preferred_element_type=jnp.float32)
    h = jnp.maximum(h + b1_ref[...], 0.0)
    h = jnp.maximum(
        jnp.dot(h, w2_ref[...], preferred_element_type=jnp.float32)
        + b2_ref[...], 0.0)
    o = jnp.dot(h, w3_ref[...], preferred_element_type=jnp.float32) + b3_ref[...]
    out_ref[...] = jnp.maximum(o[:, 0], 0.0)


def _mlp(u_rows, m_rows, W1, b1, W2, b2, W3, b3):
    w1u, w1m = W1[:_E], W1[_E:]
    grid = _B // _MLP_BLK
    row_spec = pl.BlockSpec((_MLP_BLK, 128), lambda i: (i, 0))

    def full(shape):
        return pl.BlockSpec(shape, lambda i: (0, 0))

    return pl.pallas_call(
        _mlp_body,
        grid=(grid,),
        in_specs=[row_spec, row_spec, full((_E, 64)), full((_E, 64)),
                  full((1, 64)), full((64, 16)), full((1, 16)),
                  full((16, 1)), full((1, 1))],
        out_specs=pl.BlockSpec((_MLP_BLK,), lambda i: (i,)),
        out_shape=jax.ShapeDtypeStruct((_B,), jnp.float32),
    )(u_rows, m_rows, w1u, w1m, b1.reshape(1, -1), W2, b2.reshape(1, -1),
      W3, b3.reshape(1, -1))


def kernel(users, movies, emb_users, emb_movies, W1, b1, W2, b2, W3, b3):
    users = users.astype(jnp.int32)
    movies = movies.astype(jnp.int32)
    ut = emb_users.T
    mt = emb_movies.T
    utail = ut[:, _BU * 128:]
    mtail = mt[:, _BM * 128:]
    u_rows, m_rows = _sc_gather(users, movies, ut, mt, utail, mtail)
    return _mlp(u_rows, m_rows, W1, b1, W2, b2, W3, b3)
